# baseline (speedup 1.0000x reference)
import functools

import jax
import jax.numpy as jnp
from jax import lax
from jax.experimental import pallas as pl
from jax.experimental.pallas import tpu as pltpu
from jax.experimental.pallas import tpu_sc as plsc

N, P, L, D = 1024, 2048, 75, 3
Q = 15
LANES = 16
ROW = 128
NW = 32
BPW = N // NW
CHUNKS = P // LANES

_mesh = plsc.VectorSubcoreMesh(core_axis_name="c", subcore_axis_name="s")


@functools.partial(
    pl.kernel,
    out_type=jax.ShapeDtypeStruct((N, 3, ROW), jnp.float32),
    mesh=_mesh,
    compiler_params=pltpu.CompilerParams(needs_layout_passes=False),
    scratch_types=[
        pltpu.VMEM((3, P), jnp.float32),
        pltpu.VMEM((2 * Q, ROW), jnp.float32),
        pltpu.VMEM((4, ROW), jnp.float32),
        pltpu.VMEM((3, ROW), jnp.float32),
        pltpu.VMEM((ROW,), jnp.float32),
        pltpu.VMEM((ROW,), jnp.int32),
    ],
)
def _sc_match(pts_hbm, qsp_hbm, aux_hbm, out_hbm, pts_v, qsp_v, aux_v, res_v,
              red_f, red_i):
    wid = lax.axis_index("s") * 2 + lax.axis_index("c")
    lane = lax.iota(jnp.int32, LANES)

    def batch_body(b, _):
        n = wid * BPW + b
        pltpu.sync_copy(pts_hbm.at[n], pts_v)
        pltpu.sync_copy(qsp_hbm.at[n], qsp_v)
        pltpu.sync_copy(aux_hbm.at[n], aux_v)

        idxs = jnp.zeros((LANES,), jnp.int32)
        for q in range(Q):
            qxv = qsp_v[2 * q, pl.ds(0, LANES)]
            qyv = qsp_v[2 * q + 1, pl.ds(0, LANES)]

            def chunk(i, c, qxv=qxv, qyv=qyv):
                vmin, vidx = c
                x = pts_v[0, pl.ds(i * LANES, LANES)]
                y = pts_v[1, pl.ds(i * LANES, LANES)]
                dx = x - qxv
                dy = y - qyv
                d = dx * dx + dy * dy
                cond = d < vmin
                cidx = i * LANES + lane
                return (jnp.where(cond, d, vmin), jnp.where(cond, cidx, vidx))

            vmin, vidx = lax.fori_loop(
                0, CHUNKS, chunk,
                (jnp.full((LANES,), jnp.inf, jnp.float32),
                 jnp.zeros((LANES,), jnp.int32)))

            for k in (8, 4, 2, 1):
                red_f[pl.ds(0, LANES)] = vmin
                red_i[pl.ds(0, LANES)] = vidx
                perm = lane ^ k
                m2 = plsc.load_gather(red_f, [perm])
                i2 = plsc.load_gather(red_i, [perm])
                c = (m2 < vmin) | ((m2 == vmin) & (i2 < vidx))
                vmin = jnp.where(c, m2, vmin)
                vidx = jnp.where(c, i2, vidx)
            idxs = jnp.where(lane == q, vidx, idxs)

        zero = jnp.zeros((LANES,), jnp.int32)
        gx = plsc.load_gather(pts_v, [zero, idxs])
        gy = plsc.load_gather(pts_v, [zero + 1, idxs])
        gz = plsc.load_gather(pts_v, [zero + 2, idxs])
        use = aux_v[3, pl.ds(0, LANES)] != 0.0
        res_v[0, pl.ds(0, LANES)] = jnp.where(use, gx, aux_v[0, pl.ds(0, LANES)])
        res_v[1, pl.ds(0, LANES)] = jnp.where(use, gy, aux_v[1, pl.ds(0, LANES)])
        res_v[2, pl.ds(0, LANES)] = jnp.where(use, gz, aux_v[2, pl.ds(0, LANES)])
        pltpu.sync_copy(res_v, out_hbm.at[n])
        return 0

    lax.fori_loop(0, BPW, batch_body, 0)


@jax.jit
def kernel(lmks_pred, cntr_pts, cntr_mask, lmks_real):
    ptsT = cntr_pts.transpose(0, 2, 1)
    qxy = lmks_real[:, :Q, :2]
    qsp = jnp.broadcast_to(
        qxy.reshape(N, 2 * Q, 1), (N, 2 * Q, ROW))
    lp = lmks_pred[:, :Q, :].transpose(0, 2, 1)
    lp = jnp.pad(lp, ((0, 0), (0, 0), (0, ROW - Q)))
    maskf = jnp.broadcast_to(
        cntr_mask.astype(jnp.float32)[:, None, None], (N, 1, ROW))
    aux = jnp.concatenate([lp, maskf], axis=1)

    out3 = _sc_match(ptsT, qsp, aux)
    cntr = out3.transpose(0, 2, 1)[:, :Q, :]
    return jnp.concatenate([cntr, lmks_pred[:, Q:, :]], axis=1)
